# baseline (speedup 1.0000x reference)
import functools

import jax
import jax.numpy as jnp
from jax import lax
from jax.experimental import pallas as pl
from jax.experimental.pallas import tpu as pltpu
from jax.experimental.pallas import tpu_sc as plsc

N_NODES = 10000
IN_DIM = 128
OUT_DIM = 128
N_EDGES = 320000

NC, NS, L = 2, 16, 16
NW = NC * NS
IDXW = 128
ACC_ROWS = 10240
TILE_ACC = ACC_ROWS // NS
EDGES_PAD = NW * 10240
ROWS_PER_W = EDGES_PAD // NW // IDXW
SUB = 2
SUBW = IDXW // SUB
GR = 8
GROUPS = ROWS_PER_W // GR
QSTEPS = GROUPS // 2
OUT_CHUNK = 200
NCHUNK = N_NODES // OUT_CHUNK


@functools.partial(
    pl.kernel,
    out_type=[jax.ShapeDtypeStruct((N_NODES, IN_DIM), jnp.float32),
              jax.ShapeDtypeStruct((N_NODES, IN_DIM), jnp.float32)],
    mesh=plsc.VectorSubcoreMesh(core_axis_name="c", subcore_axis_name="s"),
    scratch_types=[
        pltpu.VMEM((2, IDXW, IN_DIM), jnp.float32),
        pltpu.VMEM((2, GR, IDXW), jnp.int32),
        pltpu.VMEM((2, GR, IDXW), jnp.int32),
        pltpu.VMEM_SHARED((ACC_ROWS, IN_DIM), jnp.float32),
        pltpu.SemaphoreType.DMA,
        pltpu.SemaphoreType.DMA,
        pltpu.SemaphoreType.DMA,
        pltpu.SemaphoreType.DMA,
    ],
)
def _edge_scatter(x_hbm, src_hbm, dst_hbm, p0_hbm, p1_hbm,
                  rows, sidx, didx, acc, gsem, gsem2, ssem, isem):
    gsems = (gsem, gsem2)
    cid = lax.axis_index("c")
    sid = lax.axis_index("s")
    wid = cid * NS + sid

    z16 = jnp.zeros((L,), jnp.float32)

    def zrow(i, carry):
        for j in range(IN_DIM // L):
            rows[0, i, pl.ds(j * L, L)] = z16
        return carry

    lax.fori_loop(0, IDXW, zrow, 0)

    zc = [pltpu.async_copy(rows.at[0],
                           acc.at[pl.ds(sid * TILE_ACC + g * IDXW, IDXW)],
                           gsem)
          for g in range(TILE_ACC // IDXW)]
    for c in zc:
        c.wait()
    plsc.subcore_barrier()

    base = wid * ROWS_PER_W

    PK = 2 * GR

    def idx_fire(q, p):
        rb = base + (2 * q + p) * GR
        pltpu.async_copy(src_hbm.at[pl.ds(rb, GR)], sidx.at[p], isem)
        pltpu.async_copy(dst_hbm.at[pl.ds(rb, GR)], didx.at[p], isem)

    def idx_wait():
        pltpu.make_async_copy(src_hbm.at[pl.ds(base, GR)],
                              sidx.at[0], isem).wait()

    idx_fire(0, 0)
    idx_fire(0, 1)

    def super_step(q, carry):
        for _ in range(4):
            idx_wait()

        def fg(k):
            p, r = (0, k) if k < GR else (1, k - GR)
            b = k % 2
            for h in range(SUB):
                pltpu.async_copy(
                    x_hbm.at[sidx.at[p, r, pl.ds(h * SUBW, SUBW)]],
                    rows.at[b, pl.ds(h * SUBW, SUBW)], gsems[b])

        def wg(k):
            p, r = (0, k) if k < GR else (1, k - GR)
            b = k % 2
            pltpu.make_async_copy(x_hbm.at[sidx.at[p, r]],
                                  rows.at[b], gsems[b]).wait()

        def fs(k):
            p, r = (0, k) if k < GR else (1, k - GR)
            return pltpu.async_copy(rows.at[k % 2],
                                    acc.at[didx.at[p, r]], ssem, add=True)

        fg(0)
        s = {}
        for k in range(PK):
            if k + 1 < PK:
                if k >= 1:
                    s[k - 1].wait()
                fg(k + 1)
            if k == GR + 1:
                @pl.when(q < QSTEPS - 1)
                def _():
                    idx_fire(q + 1, 0)
            wg(k)
            s[k] = fs(k)
        s[PK - 2].wait()
        s[PK - 1].wait()

        @pl.when(q < QSTEPS - 1)
        def _():
            idx_fire(q + 1, 1)

        return carry

    lax.fori_loop(0, QSTEPS, super_step, 0)
    plsc.subcore_barrier()

    def copy_chunk(c):
        @pl.when(cid == 0)
        def _():
            pltpu.async_copy(acc.at[pl.ds(c * OUT_CHUNK, OUT_CHUNK)],
                             p0_hbm.at[pl.ds(c * OUT_CHUNK, OUT_CHUNK)],
                             gsem)

        @pl.when(cid == 1)
        def _():
            pltpu.async_copy(acc.at[pl.ds(c * OUT_CHUNK, OUT_CHUNK)],
                             p1_hbm.at[pl.ds(c * OUT_CHUNK, OUT_CHUNK)],
                             gsem)

    NK = (NCHUNK + NS - 1) // NS
    for k in range(NK):
        c = sid + k * NS

        @pl.when(c < NCHUNK)
        def _(c=c):
            copy_chunk(c)

    for k in range(NK):
        c = sid + k * NS

        @pl.when(c < NCHUNK)
        def _():
            pltpu.make_async_copy(
                acc.at[pl.ds(0, OUT_CHUNK)],
                p0_hbm.at[pl.ds(0, OUT_CHUNK)], gsem).wait()


BLK = 2000


def _combine_body(x_ref, w_ref, a_ref, b_ref, o_ref):
    o_ref[...] = (jnp.dot(x_ref[...], w_ref[...],
                          preferred_element_type=jnp.float32)
                  + a_ref[...] + b_ref[...])


_combine = pl.pallas_call(
    _combine_body,
    grid=(N_NODES // BLK,),
    in_specs=[pl.BlockSpec((BLK, IN_DIM), lambda i: (i, 0)),
              pl.BlockSpec((IN_DIM, OUT_DIM), lambda i: (0, 0)),
              pl.BlockSpec((BLK, IN_DIM), lambda i: (i, 0)),
              pl.BlockSpec((BLK, IN_DIM), lambda i: (i, 0))],
    out_specs=pl.BlockSpec((BLK, OUT_DIM), lambda i: (i, 0)),
    out_shape=jax.ShapeDtypeStruct((N_NODES, OUT_DIM), jnp.float32),
)


def kernel(x, edge_index, loop_weight):
    src = edge_index[0].astype(jnp.int32)
    dst = edge_index[1].astype(jnp.int32)
    pad = EDGES_PAD - N_EDGES
    pad_ar = jnp.arange(pad, dtype=jnp.int32)
    src_p = jnp.concatenate([src, pad_ar % N_NODES])
    dst_p = jnp.concatenate([dst, N_NODES + pad_ar % (ACC_ROWS - N_NODES)])
    src2d = src_p.reshape(EDGES_PAD // IDXW, IDXW)
    dst2d = dst_p.reshape(EDGES_PAD // IDXW, IDXW)
    p0, p1 = _edge_scatter(x, src2d, dst2d)
    return _combine(x, loop_weight, p0, p1)


# speedup vs baseline: 1.0201x; 1.0201x over previous
import functools

import jax
import jax.numpy as jnp
from jax import lax
from jax.experimental import pallas as pl
from jax.experimental.pallas import tpu as pltpu
from jax.experimental.pallas import tpu_sc as plsc

N_NODES = 10000
IN_DIM = 128
OUT_DIM = 128
N_EDGES = 320000

NC, NS, L = 2, 16, 16
NW = NC * NS
IDXW = 128
ACC_ROWS = 10240
TILE_ACC = ACC_ROWS // NS
EDGES_PAD = NW * 10240
ROWS_PER_W = EDGES_PAD // NW // IDXW
SUB = 2
SUBW = IDXW // SUB
GR = 8
GROUPS = ROWS_PER_W // GR
QSTEPS = GROUPS // 2
OUT_CHUNK = 200
NCHUNK = N_NODES // OUT_CHUNK


@functools.partial(
    pl.kernel,
    out_type=[jax.ShapeDtypeStruct((N_NODES, IN_DIM), jnp.float32),
              jax.ShapeDtypeStruct((N_NODES, IN_DIM), jnp.float32)],
    mesh=plsc.VectorSubcoreMesh(core_axis_name="c", subcore_axis_name="s"),
    scratch_types=[
        pltpu.VMEM((2, IDXW, IN_DIM), jnp.float32),
        pltpu.VMEM((2, GR, IDXW), jnp.int32),
        pltpu.VMEM((2, GR, IDXW), jnp.int32),
        pltpu.VMEM_SHARED((ACC_ROWS, IN_DIM), jnp.float32),
        pltpu.SemaphoreType.DMA,
        pltpu.SemaphoreType.DMA,
        pltpu.SemaphoreType.DMA,
    ],
)
def _edge_scatter(x_hbm, src_hbm, dst_hbm, p0_hbm, p1_hbm,
                  rows, sidx, didx, acc, gsem, ssem, isem):
    cid = lax.axis_index("c")
    sid = lax.axis_index("s")
    wid = cid * NS + sid

    z16 = jnp.zeros((L,), jnp.float32)

    def zrow(i, carry):
        for j in range(IN_DIM // L):
            rows[0, i, pl.ds(j * L, L)] = z16
        return carry

    lax.fori_loop(0, IDXW, zrow, 0)

    zc = [pltpu.async_copy(rows.at[0],
                           acc.at[pl.ds(sid * TILE_ACC + g * IDXW, IDXW)],
                           gsem)
          for g in range(TILE_ACC // IDXW)]
    for c in zc:
        c.wait()
    plsc.subcore_barrier()

    base = wid * ROWS_PER_W

    PK = 2 * GR

    def idx_fire(q, p):
        rb = base + (2 * q + p) * GR
        pltpu.async_copy(src_hbm.at[pl.ds(rb, GR)], sidx.at[p], isem)
        pltpu.async_copy(dst_hbm.at[pl.ds(rb, GR)], didx.at[p], isem)

    def idx_wait():
        pltpu.make_async_copy(src_hbm.at[pl.ds(base, GR)],
                              sidx.at[0], isem).wait()

    idx_fire(0, 0)
    idx_fire(0, 1)

    def super_step(q, carry):
        for _ in range(4):
            idx_wait()

        def fg(k):
            p, r = (0, k) if k < GR else (1, k - GR)
            b = k % 2
            return [pltpu.async_copy(
                x_hbm.at[sidx.at[p, r, pl.ds(h * SUBW, SUBW)]],
                rows.at[b, pl.ds(h * SUBW, SUBW)], gsem)
                for h in range(SUB)]

        def fs(k):
            p, r = (0, k) if k < GR else (1, k - GR)
            return pltpu.async_copy(rows.at[k % 2],
                                    acc.at[didx.at[p, r]], ssem, add=True)

        g = {0: fg(0)}
        s = {}
        for k in range(PK):
            if k + 1 < PK:
                if k >= 1:
                    s[k - 1].wait()
                g[k + 1] = fg(k + 1)
            if k == GR + 1:
                @pl.when(q < QSTEPS - 1)
                def _():
                    idx_fire(q + 1, 0)
            for c in g[k]:
                c.wait()
            s[k] = fs(k)
        s[PK - 2].wait()
        s[PK - 1].wait()

        @pl.when(q < QSTEPS - 1)
        def _():
            idx_fire(q + 1, 1)

        return carry

    lax.fori_loop(0, QSTEPS, super_step, 0)
    plsc.subcore_barrier()

    def copy_chunk(c):
        @pl.when(cid == 0)
        def _():
            pltpu.async_copy(acc.at[pl.ds(c * OUT_CHUNK, OUT_CHUNK)],
                             p0_hbm.at[pl.ds(c * OUT_CHUNK, OUT_CHUNK)],
                             gsem)

        @pl.when(cid == 1)
        def _():
            pltpu.async_copy(acc.at[pl.ds(c * OUT_CHUNK, OUT_CHUNK)],
                             p1_hbm.at[pl.ds(c * OUT_CHUNK, OUT_CHUNK)],
                             gsem)

    NK = (NCHUNK + NS - 1) // NS
    for k in range(NK):
        c = sid + k * NS

        @pl.when(c < NCHUNK)
        def _(c=c):
            copy_chunk(c)

    for k in range(NK):
        c = sid + k * NS

        @pl.when(c < NCHUNK)
        def _():
            pltpu.make_async_copy(
                acc.at[pl.ds(0, OUT_CHUNK)],
                p0_hbm.at[pl.ds(0, OUT_CHUNK)], gsem).wait()


BLK = 2000


def _combine_body(x_ref, w_ref, a_ref, b_ref, o_ref):
    o_ref[...] = (jnp.dot(x_ref[...], w_ref[...],
                          preferred_element_type=jnp.float32)
                  + a_ref[...] + b_ref[...])


_combine = pl.pallas_call(
    _combine_body,
    grid=(N_NODES // BLK,),
    in_specs=[pl.BlockSpec((BLK, IN_DIM), lambda i: (i, 0)),
              pl.BlockSpec((IN_DIM, OUT_DIM), lambda i: (0, 0)),
              pl.BlockSpec((BLK, IN_DIM), lambda i: (i, 0)),
              pl.BlockSpec((BLK, IN_DIM), lambda i: (i, 0))],
    out_specs=pl.BlockSpec((BLK, OUT_DIM), lambda i: (i, 0)),
    out_shape=jax.ShapeDtypeStruct((N_NODES, OUT_DIM), jnp.float32),
)


def kernel(x, edge_index, loop_weight):
    src = edge_index[0].astype(jnp.int32)
    dst = edge_index[1].astype(jnp.int32)
    pad = EDGES_PAD - N_EDGES
    pad_ar = jnp.arange(pad, dtype=jnp.int32)
    src_p = jnp.concatenate([src, pad_ar % N_NODES])
    dst_p = jnp.concatenate([dst, N_NODES + pad_ar % (ACC_ROWS - N_NODES)])
    src2d = src_p.reshape(EDGES_PAD // IDXW, IDXW)
    dst2d = dst_p.reshape(EDGES_PAD // IDXW, IDXW)
    p0, p1 = _edge_scatter(x, src2d, dst2d)
    return _combine(x, loop_weight, p0, p1)


# speedup vs baseline: 1.0261x; 1.0059x over previous
import functools

import jax
import jax.numpy as jnp
from jax import lax
from jax.experimental import pallas as pl
from jax.experimental.pallas import tpu as pltpu
from jax.experimental.pallas import tpu_sc as plsc

N_NODES = 10000
IN_DIM = 128
OUT_DIM = 128
N_EDGES = 320000

NC, NS, L = 2, 16, 16
NW = NC * NS
IDXW = 128
ACC_ROWS = 10240
TILE_ACC = ACC_ROWS // NS
EDGES_PAD = NW * 10240
ROWS_PER_W = EDGES_PAD // NW // IDXW
SUB = 2
SUBW = IDXW // SUB
GR = 8
GROUPS = ROWS_PER_W // GR
QSTEPS = GROUPS // 2
OUT_CHUNK = 400
NCHUNK = N_NODES // OUT_CHUNK


@functools.partial(
    pl.kernel,
    out_type=[jax.ShapeDtypeStruct((N_NODES, IN_DIM), jnp.float32),
              jax.ShapeDtypeStruct((N_NODES, IN_DIM), jnp.float32)],
    mesh=plsc.VectorSubcoreMesh(core_axis_name="c", subcore_axis_name="s"),
    scratch_types=[
        pltpu.VMEM((2, IDXW, IN_DIM), jnp.float32),
        pltpu.VMEM((2, GR, IDXW), jnp.int32),
        pltpu.VMEM((2, GR, IDXW), jnp.int32),
        pltpu.VMEM_SHARED((ACC_ROWS, IN_DIM), jnp.float32),
        pltpu.SemaphoreType.DMA,
        pltpu.SemaphoreType.DMA,
        pltpu.SemaphoreType.DMA,
    ],
)
def _edge_scatter(x_hbm, src_hbm, dst_hbm, p0_hbm, p1_hbm,
                  rows, sidx, didx, acc, gsem, ssem, isem):
    cid = lax.axis_index("c")
    sid = lax.axis_index("s")
    wid = cid * NS + sid
    base = wid * ROWS_PER_W

    def idx_fire(q, p):
        rb = base + (2 * q + p) * GR
        pltpu.async_copy(src_hbm.at[pl.ds(rb, GR)], sidx.at[p], isem)
        pltpu.async_copy(dst_hbm.at[pl.ds(rb, GR)], didx.at[p], isem)

    def idx_wait():
        pltpu.make_async_copy(src_hbm.at[pl.ds(base, GR)],
                              sidx.at[0], isem).wait()

    idx_fire(0, 0)
    idx_fire(0, 1)

    z16 = jnp.zeros((L,), jnp.float32)

    def zrow(i, carry):
        for j in range(IN_DIM // L):
            rows[0, i, pl.ds(j * L, L)] = z16
        return carry

    lax.fori_loop(0, IDXW, zrow, 0)

    zc = [pltpu.async_copy(rows.at[0],
                           acc.at[pl.ds(sid * TILE_ACC + g * IDXW, IDXW)],
                           gsem)
          for g in range(TILE_ACC // IDXW)]
    for c in zc:
        c.wait()
    plsc.subcore_barrier()

    PK = 2 * GR

    def super_step(q, carry):
        for _ in range(4):
            idx_wait()

        def fg(k):
            p, r = (0, k) if k < GR else (1, k - GR)
            b = k % 2
            return [pltpu.async_copy(
                x_hbm.at[sidx.at[p, r, pl.ds(h * SUBW, SUBW)]],
                rows.at[b, pl.ds(h * SUBW, SUBW)], gsem)
                for h in range(SUB)]

        def fs(k):
            p, r = (0, k) if k < GR else (1, k - GR)
            return pltpu.async_copy(rows.at[k % 2],
                                    acc.at[didx.at[p, r]], ssem, add=True)

        g = {0: fg(0)}
        s = {}
        for k in range(PK):
            if k + 1 < PK:
                if k >= 1:
                    s[k - 1].wait()
                g[k + 1] = fg(k + 1)
            if k == GR + 1:
                @pl.when(q < QSTEPS - 1)
                def _():
                    idx_fire(q + 1, 0)
            for c in g[k]:
                c.wait()
            s[k] = fs(k)
        s[PK - 2].wait()
        s[PK - 1].wait()

        @pl.when(q < QSTEPS - 1)
        def _():
            idx_fire(q + 1, 1)

        return carry

    lax.fori_loop(0, QSTEPS, super_step, 0)
    plsc.subcore_barrier()

    def copy_chunk(c):
        @pl.when(cid == 0)
        def _():
            pltpu.async_copy(acc.at[pl.ds(c * OUT_CHUNK, OUT_CHUNK)],
                             p0_hbm.at[pl.ds(c * OUT_CHUNK, OUT_CHUNK)],
                             gsem)

        @pl.when(cid == 1)
        def _():
            pltpu.async_copy(acc.at[pl.ds(c * OUT_CHUNK, OUT_CHUNK)],
                             p1_hbm.at[pl.ds(c * OUT_CHUNK, OUT_CHUNK)],
                             gsem)

    NK = (NCHUNK + NS - 1) // NS
    for k in range(NK):
        c = sid + k * NS

        @pl.when(c < NCHUNK)
        def _(c=c):
            copy_chunk(c)

    for k in range(NK):
        c = sid + k * NS

        @pl.when(c < NCHUNK)
        def _():
            pltpu.make_async_copy(
                acc.at[pl.ds(0, OUT_CHUNK)],
                p0_hbm.at[pl.ds(0, OUT_CHUNK)], gsem).wait()


BLK = 2000


def _combine_body(x_ref, w_ref, a_ref, b_ref, o_ref):
    o_ref[...] = (jnp.dot(x_ref[...], w_ref[...],
                          preferred_element_type=jnp.float32)
                  + a_ref[...] + b_ref[...])


_combine = pl.pallas_call(
    _combine_body,
    grid=(N_NODES // BLK,),
    in_specs=[pl.BlockSpec((BLK, IN_DIM), lambda i: (i, 0)),
              pl.BlockSpec((IN_DIM, OUT_DIM), lambda i: (0, 0)),
              pl.BlockSpec((BLK, IN_DIM), lambda i: (i, 0)),
              pl.BlockSpec((BLK, IN_DIM), lambda i: (i, 0))],
    out_specs=pl.BlockSpec((BLK, OUT_DIM), lambda i: (i, 0)),
    out_shape=jax.ShapeDtypeStruct((N_NODES, OUT_DIM), jnp.float32),
)


def kernel(x, edge_index, loop_weight):
    src = edge_index[0].astype(jnp.int32)
    dst = edge_index[1].astype(jnp.int32)
    pad = EDGES_PAD - N_EDGES
    pad_ar = jnp.arange(pad, dtype=jnp.int32)
    src_p = jnp.concatenate([src, pad_ar % N_NODES])
    dst_p = jnp.concatenate([dst, N_NODES + pad_ar % (ACC_ROWS - N_NODES)])
    src2d = src_p.reshape(EDGES_PAD // IDXW, IDXW)
    dst2d = dst_p.reshape(EDGES_PAD // IDXW, IDXW)
    p0, p1 = _edge_scatter(x, src2d, dst2d)
    return _combine(x, loop_weight, p0, p1)
